# baseline (speedup 1.0000x reference)
import functools

import jax
import jax.numpy as jnp
from jax import lax
from jax.experimental import pallas as pl
from jax.experimental.pallas import tpu as pltpu
from jax.experimental.pallas import tpu_sc as plsc

MAXN = 256
LANES = 16


@functools.cache
def _build_lookup(n):
    info = plsc.get_sparse_core_info()
    nw = info.num_cores * info.num_subcores
    assert n % (nw * LANES) == 0
    per_w = n // nw
    chunk = 4096
    while per_w % chunk:
        chunk //= 2
    nchunks = per_w // chunk
    mesh = plsc.VectorSubcoreMesh(core_axis_name="c", subcore_axis_name="s")

    @functools.partial(
        pl.kernel,
        mesh=mesh,
        compiler_params=pltpu.CompilerParams(needs_layout_passes=False),
        out_type=jax.ShapeDtypeStruct((2 * n,), jnp.float32),
        scratch_types=[
            pltpu.VMEM((MAXN,), jnp.float32),
            pltpu.VMEM((MAXN,), jnp.float32),
            pltpu.VMEM((chunk,), jnp.int32),
            pltpu.VMEM((2 * chunk,), jnp.float32),
        ],
    )
    def lookup(tab_r_hbm, tab_i_hbm, pos_hbm, out_hbm,
               tab_r, tab_i, pos_v, ri_v):
        wid = lax.axis_index("s") * info.num_cores + lax.axis_index("c")
        base = wid * per_w
        pltpu.sync_copy(tab_r_hbm, tab_r)
        pltpu.sync_copy(tab_i_hbm, tab_i)
        lane2 = lax.iota(jnp.int32, LANES) * 2

        def chunk_body(g, carry):
            off = base + g * chunk
            pltpu.sync_copy(pos_hbm.at[pl.ds(off, chunk)], pos_v)

            def body(i, c):
                idx = pos_v[pl.ds(i * LANES, LANES)] & (MAXN - 1)
                re = plsc.load_gather(tab_r, [idx])
                im = plsc.load_gather(tab_i, [idx])
                b2 = i * (2 * LANES) + lane2
                plsc.store_scatter(ri_v, [b2], re)
                plsc.store_scatter(ri_v, [b2 + 1], im)
                return c

            lax.fori_loop(0, chunk // LANES, body, 0)
            pltpu.sync_copy(ri_v, out_hbm.at[pl.ds(2 * off, 2 * chunk)])
            return carry

        lax.fori_loop(0, nchunks, chunk_body, 0)

    return lookup


def kernel(pe, pos):
    shape = pos.shape
    n = pos.size
    tab_r = jnp.real(pe).astype(jnp.float32)
    tab_i = jnp.imag(pe).astype(jnp.float32)
    out_ri = _build_lookup(n)(tab_r, tab_i, pos.reshape(n))
    return lax.complex(out_ri[0::2], out_ri[1::2]).reshape(shape)


# speedup vs baseline: 3.3646x; 3.3646x over previous
import functools

import jax
import jax.numpy as jnp
from jax import lax
from jax.experimental import pallas as pl
from jax.experimental.pallas import tpu as pltpu
from jax.experimental.pallas import tpu_sc as plsc

MAXN = 256
LANES = 16


@functools.cache
def _build_lookup(n):
    info = plsc.get_sparse_core_info()
    nw = info.num_cores * info.num_subcores
    assert n % (nw * LANES) == 0
    per_w = n // nw
    chunk = 4096
    while per_w % chunk:
        chunk //= 2
    nchunks = per_w // chunk
    mesh = plsc.VectorSubcoreMesh(core_axis_name="c", subcore_axis_name="s")

    @functools.partial(
        pl.kernel,
        mesh=mesh,
        compiler_params=pltpu.CompilerParams(needs_layout_passes=False),
        out_type=[
            jax.ShapeDtypeStruct((n,), jnp.float32),
            jax.ShapeDtypeStruct((n,), jnp.float32),
        ],
        scratch_types=[
            pltpu.VMEM((MAXN,), jnp.float32),
            pltpu.VMEM((MAXN,), jnp.float32),
            pltpu.VMEM((chunk,), jnp.int32),
            pltpu.VMEM((chunk,), jnp.float32),
            pltpu.VMEM((chunk,), jnp.float32),
        ],
    )
    def lookup(tab_r_hbm, tab_i_hbm, pos_hbm, out_r_hbm, out_i_hbm,
               tab_r, tab_i, pos_v, re_v, im_v):
        wid = lax.axis_index("s") * info.num_cores + lax.axis_index("c")
        base = wid * per_w
        pltpu.sync_copy(tab_r_hbm, tab_r)
        pltpu.sync_copy(tab_i_hbm, tab_i)

        def chunk_body(g, carry):
            off = base + g * chunk
            pltpu.sync_copy(pos_hbm.at[pl.ds(off, chunk)], pos_v)

            def body(i, c):
                idx = pos_v[pl.ds(i * LANES, LANES)] & (MAXN - 1)
                re_v[pl.ds(i * LANES, LANES)] = plsc.load_gather(tab_r, [idx])
                im_v[pl.ds(i * LANES, LANES)] = plsc.load_gather(tab_i, [idx])
                return c

            lax.fori_loop(0, chunk // LANES, body, 0)
            pltpu.sync_copy(re_v, out_r_hbm.at[pl.ds(off, chunk)])
            pltpu.sync_copy(im_v, out_i_hbm.at[pl.ds(off, chunk)])
            return carry

        lax.fori_loop(0, nchunks, chunk_body, 0)

    return lookup


def kernel(pe, pos):
    shape = pos.shape
    n = pos.size
    tab_r = jnp.real(pe).astype(jnp.float32)
    tab_i = jnp.imag(pe).astype(jnp.float32)
    out_r, out_i = _build_lookup(n)(tab_r, tab_i, pos.reshape(n))
    return lax.complex(out_r.reshape(shape), out_i.reshape(shape))


# speedup vs baseline: 4.2825x; 1.2728x over previous
import functools

import jax
import jax.numpy as jnp
from jax import lax
from jax.experimental import pallas as pl
from jax.experimental.pallas import tpu as pltpu
from jax.experimental.pallas import tpu_sc as plsc

MAXN = 256
LANES = 16


@functools.cache
def _build_lookup(nb, ns):
    info = plsc.get_sparse_core_info()
    nw = info.num_cores * info.num_subcores
    assert nb % nw == 0
    rows_w = nb // nw
    cb = rows_w
    while cb * ns * 3 > 120_000 or rows_w % cb:
        cb //= 2
    assert cb % LANES == 0 and rows_w % cb == 0
    nchunks = rows_w // cb
    kb = cb // LANES
    mesh = plsc.VectorSubcoreMesh(core_axis_name="c", subcore_axis_name="s")

    @functools.partial(
        pl.kernel,
        mesh=mesh,
        compiler_params=pltpu.CompilerParams(needs_layout_passes=False),
        out_type=[
            jax.ShapeDtypeStruct((ns, nb), jnp.float32),
            jax.ShapeDtypeStruct((ns, nb), jnp.float32),
        ],
        scratch_types=[
            pltpu.VMEM((MAXN,), jnp.float32),
            pltpu.VMEM((MAXN,), jnp.float32),
            pltpu.VMEM((cb * ns,), jnp.int32),
            pltpu.VMEM((ns, cb), jnp.float32),
            pltpu.VMEM((ns, cb), jnp.float32),
        ],
    )
    def lookup(tab_r_hbm, tab_i_hbm, pos_hbm, out_r_hbm, out_i_hbm,
               tab_r, tab_i, pos_v, re_v, im_v):
        wid = lax.axis_index("s") * info.num_cores + lax.axis_index("c")
        base = wid * rows_w
        pltpu.sync_copy(tab_r_hbm, tab_r)
        pltpu.sync_copy(tab_i_hbm, tab_i)
        lane_s = lax.iota(jnp.int32, LANES) * ns

        def chunk_body(g, carry):
            b0 = base + g * cb
            pltpu.sync_copy(pos_hbm.at[pl.ds(b0 * ns, cb * ns)], pos_v)

            def body(i, c):
                s = i // kb
                k = i % kb
                p = plsc.load_gather(pos_v, [lane_s + (k * LANES * ns + s)])
                idx = p & (MAXN - 1)
                re_v[s, pl.ds(k * LANES, LANES)] = plsc.load_gather(tab_r, [idx])
                im_v[s, pl.ds(k * LANES, LANES)] = plsc.load_gather(tab_i, [idx])
                return c

            lax.fori_loop(0, ns * kb, body, 0)
            pltpu.sync_copy(re_v, out_r_hbm.at[:, pl.ds(b0, cb)])
            pltpu.sync_copy(im_v, out_i_hbm.at[:, pl.ds(b0, cb)])
            return carry

        lax.fori_loop(0, nchunks, chunk_body, 0)

    return lookup


def kernel(pe, pos):
    nb, ns = pos.shape
    tab_r = jnp.real(pe).astype(jnp.float32)
    tab_i = jnp.imag(pe).astype(jnp.float32)
    out_r, out_i = _build_lookup(nb, ns)(tab_r, tab_i, pos.reshape(nb * ns))
    return lax.complex(out_r, out_i).T


# speedup vs baseline: 5.5217x; 1.2893x over previous
import functools

import jax
import jax.numpy as jnp
from jax import lax
from jax.experimental import pallas as pl
from jax.experimental.pallas import tpu as pltpu
from jax.experimental.pallas import tpu_sc as plsc

MAXN = 256
LANES = 16


@functools.cache
def _build_lookup(nb, ns):
    info = plsc.get_sparse_core_info()
    nw = info.num_cores * info.num_subcores
    assert nb % nw == 0
    rows_w = nb // nw
    cb = rows_w
    while cb * ns * 3 > 120_000 or rows_w % cb:
        cb //= 2
    assert cb % LANES == 0 and rows_w % cb == 0
    nchunks = rows_w // cb
    kb = cb // LANES
    mesh = plsc.VectorSubcoreMesh(core_axis_name="c", subcore_axis_name="s")

    @functools.partial(
        pl.kernel,
        mesh=mesh,
        compiler_params=pltpu.CompilerParams(needs_layout_passes=False),
        out_type=[
            jax.ShapeDtypeStruct((ns, nb), jnp.float32),
            jax.ShapeDtypeStruct((ns, nb), jnp.float32),
        ],
        scratch_types=[
            pltpu.VMEM((MAXN,), jnp.float32),
            pltpu.VMEM((MAXN,), jnp.float32),
            pltpu.VMEM((cb * ns,), jnp.int32),
            pltpu.VMEM((ns, cb), jnp.float32),
            pltpu.VMEM((ns, cb), jnp.float32),
        ],
    )
    def lookup(tab_r_hbm, tab_i_hbm, pos_hbm, out_r_hbm, out_i_hbm,
               tab_r, tab_i, pos_v, re_v, im_v):
        wid = lax.axis_index("s") * info.num_cores + lax.axis_index("c")
        base = wid * rows_w
        pltpu.sync_copy(tab_r_hbm, tab_r)
        pltpu.sync_copy(tab_i_hbm, tab_i)
        lane_s = lax.iota(jnp.int32, LANES) * ns

        def chunk_body(g, carry):
            b0 = base + g * cb
            pltpu.sync_copy(pos_hbm.at[pl.ds(b0 * ns, cb * ns)], pos_v)

            @plsc.parallel_loop(0, ns * kb, unroll=8)
            def body(i):
                s = i // kb
                k = i % kb
                p = plsc.load_gather(pos_v, [lane_s + (k * LANES * ns + s)])
                idx = p & (MAXN - 1)
                re_v[s, pl.ds(k * LANES, LANES)] = plsc.load_gather(tab_r, [idx])
                im_v[s, pl.ds(k * LANES, LANES)] = plsc.load_gather(tab_i, [idx])
            pltpu.sync_copy(re_v, out_r_hbm.at[:, pl.ds(b0, cb)])
            pltpu.sync_copy(im_v, out_i_hbm.at[:, pl.ds(b0, cb)])
            return carry

        lax.fori_loop(0, nchunks, chunk_body, 0)

    return lookup


def kernel(pe, pos):
    nb, ns = pos.shape
    tab_r = jnp.real(pe).astype(jnp.float32)
    tab_i = jnp.imag(pe).astype(jnp.float32)
    out_r, out_i = _build_lookup(nb, ns)(tab_r, tab_i, pos.reshape(nb * ns))
    return lax.complex(out_r, out_i).T


# speedup vs baseline: 5.5868x; 1.0118x over previous
import functools

import jax
import jax.numpy as jnp
from jax import lax
from jax.experimental import pallas as pl
from jax.experimental.pallas import tpu as pltpu
from jax.experimental.pallas import tpu_sc as plsc

MAXN = 256
LANES = 16


@functools.cache
def _build_lookup(nb, ns):
    info = plsc.get_sparse_core_info()
    nw = info.num_cores * info.num_subcores
    assert nb % nw == 0
    rows_w = nb // nw
    cb = rows_w
    while cb * ns * 5 > 130_000 or rows_w % cb or cb % 128:
        cb //= 2
    assert cb % LANES == 0 and rows_w % cb == 0
    nchunks = rows_w // cb
    kb = cb // LANES
    mesh = plsc.VectorSubcoreMesh(core_axis_name="c", subcore_axis_name="s")

    @functools.partial(
        pl.kernel,
        mesh=mesh,
        compiler_params=pltpu.CompilerParams(needs_layout_passes=False),
        out_type=[
            jax.ShapeDtypeStruct((ns, nb), jnp.float32),
            jax.ShapeDtypeStruct((ns, nb), jnp.float32),
        ],
        scratch_types=[
            pltpu.VMEM((MAXN,), jnp.float32),
            pltpu.VMEM((MAXN,), jnp.float32),
            pltpu.VMEM((cb * ns,), jnp.int32),
            [pltpu.VMEM((ns, cb), jnp.float32)] * 2,
            [pltpu.VMEM((ns, cb), jnp.float32)] * 2,
            [pltpu.SemaphoreType.DMA] * 2,
        ],
    )
    def lookup(tab_r_hbm, tab_i_hbm, pos_hbm, out_r_hbm, out_i_hbm,
               tab_r, tab_i, pos_v, re_v, im_v, osem):
        wid = lax.axis_index("s") * info.num_cores + lax.axis_index("c")
        base = wid * rows_w
        pltpu.sync_copy(tab_r_hbm, tab_r)
        pltpu.sync_copy(tab_i_hbm, tab_i)
        lane_s = lax.iota(jnp.int32, LANES) * ns

        out_dmas = [None, None]
        for g in range(nchunks):
            j = g % 2
            pv, rv, iv = pos_v, re_v[j], im_v[j]
            pltpu.sync_copy(
                pos_hbm.at[pl.ds((base + g * cb) * ns, cb * ns)], pos_v)
            if out_dmas[j] is not None:
                for d in out_dmas[j]:
                    d.wait()

            @plsc.parallel_loop(0, ns * kb, unroll=8)
            def body(i):
                s = i // kb
                k = i % kb
                p = plsc.load_gather(pv, [lane_s + (k * LANES * ns + s)])
                idx = p & (MAXN - 1)
                rv[s, pl.ds(k * LANES, LANES)] = plsc.load_gather(tab_r, [idx])
                iv[s, pl.ds(k * LANES, LANES)] = plsc.load_gather(tab_i, [idx])

            b0 = base + g * cb
            out_dmas[j] = (
                pltpu.async_copy(rv, out_r_hbm.at[:, pl.ds(b0, cb)], osem[j]),
                pltpu.async_copy(iv, out_i_hbm.at[:, pl.ds(b0, cb)], osem[j]),
            )
        for pair in out_dmas:
            if pair is not None:
                for d in pair:
                    d.wait()

    return lookup


def kernel(pe, pos):
    nb, ns = pos.shape
    tab_r = jnp.real(pe).astype(jnp.float32)
    tab_i = jnp.imag(pe).astype(jnp.float32)
    out_r, out_i = _build_lookup(nb, ns)(tab_r, tab_i, pos.reshape(nb * ns))
    return lax.complex(out_r, out_i).T


# speedup vs baseline: 5.6140x; 1.0049x over previous
import functools

import jax
import jax.numpy as jnp
from jax import lax
from jax.experimental import pallas as pl
from jax.experimental.pallas import tpu as pltpu
from jax.experimental.pallas import tpu_sc as plsc

MAXN = 256
LANES = 16


@functools.cache
def _build_lookup(nb, ns):
    info = plsc.get_sparse_core_info()
    nw = info.num_cores * info.num_subcores
    assert nb % nw == 0
    rows_w = nb // nw
    cb = rows_w
    while cb * ns * 5 > 130_000 or rows_w % cb or cb % 128:
        cb //= 2
    assert cb % LANES == 0 and rows_w % cb == 0
    nchunks = rows_w // cb
    kb = cb // LANES
    mesh = plsc.VectorSubcoreMesh(core_axis_name="c", subcore_axis_name="s")

    @functools.partial(
        pl.kernel,
        mesh=mesh,
        compiler_params=pltpu.CompilerParams(needs_layout_passes=False),
        out_type=[
            jax.ShapeDtypeStruct((ns, nb), jnp.float32),
            jax.ShapeDtypeStruct((ns, nb), jnp.float32),
        ],
        scratch_types=[
            pltpu.VMEM((MAXN,), jnp.float32),
            pltpu.VMEM((MAXN,), jnp.float32),
            pltpu.VMEM((cb * ns,), jnp.int32),
            [pltpu.VMEM((ns, cb), jnp.float32)] * 2,
            [pltpu.VMEM((ns, cb), jnp.float32)] * 2,
            [pltpu.SemaphoreType.DMA] * 2,
        ],
    )
    def lookup(tab_r_hbm, tab_i_hbm, pos_hbm, out_r_hbm, out_i_hbm,
               tab_r, tab_i, pos_v, re_v, im_v, osem):
        wid = lax.axis_index("s") * info.num_cores + lax.axis_index("c")
        base = wid * rows_w
        pltpu.sync_copy(tab_r_hbm, tab_r)
        pltpu.sync_copy(tab_i_hbm, tab_i)
        lane_s = lax.iota(jnp.int32, LANES) * ns

        out_dmas = [None, None]
        for g in range(nchunks):
            j = g % 2
            pv, rv, iv = pos_v, re_v[j], im_v[j]
            pltpu.sync_copy(
                pos_hbm.at[pl.ds((base + g * cb) * ns, cb * ns)], pos_v)
            if out_dmas[j] is not None:
                for d in out_dmas[j]:
                    d.wait()

            @plsc.parallel_loop(0, ns * kb, unroll=16)
            def body(i):
                s = i // kb
                k = i % kb
                p = plsc.load_gather(pv, [lane_s + (k * LANES * ns + s)])
                idx = p & (MAXN - 1)
                rv[s, pl.ds(k * LANES, LANES)] = plsc.load_gather(tab_r, [idx])
                iv[s, pl.ds(k * LANES, LANES)] = plsc.load_gather(tab_i, [idx])

            b0 = base + g * cb
            out_dmas[j] = (
                pltpu.async_copy(rv, out_r_hbm.at[:, pl.ds(b0, cb)], osem[j]),
                pltpu.async_copy(iv, out_i_hbm.at[:, pl.ds(b0, cb)], osem[j]),
            )
        for pair in out_dmas:
            if pair is not None:
                for d in pair:
                    d.wait()

    return lookup


def kernel(pe, pos):
    nb, ns = pos.shape
    tab_r = jnp.real(pe).astype(jnp.float32)
    tab_i = jnp.imag(pe).astype(jnp.float32)
    out_r, out_i = _build_lookup(nb, ns)(tab_r, tab_i, pos.reshape(nb * ns))
    return lax.complex(out_r, out_i).T


# speedup vs baseline: 5.6352x; 1.0038x over previous
import functools

import jax
import jax.numpy as jnp
from jax import lax
from jax.experimental import pallas as pl
from jax.experimental.pallas import tpu as pltpu
from jax.experimental.pallas import tpu_sc as plsc

MAXN = 256
LANES = 16


@functools.cache
def _build_lookup(nb, ns):
    info = plsc.get_sparse_core_info()
    nw = info.num_cores * info.num_subcores
    assert nb % nw == 0
    rows_w = nb // nw
    cb = rows_w
    while cb * ns * 5 > 130_000 or rows_w % cb or cb % 128:
        cb //= 2
    assert cb % LANES == 0 and rows_w % cb == 0
    nchunks = rows_w // cb
    hb = cb // 2
    kbh = hb // LANES
    mesh = plsc.VectorSubcoreMesh(core_axis_name="c", subcore_axis_name="s")

    @functools.partial(
        pl.kernel,
        mesh=mesh,
        compiler_params=pltpu.CompilerParams(needs_layout_passes=False),
        out_type=[
            jax.ShapeDtypeStruct((ns, nb), jnp.float32),
            jax.ShapeDtypeStruct((ns, nb), jnp.float32),
        ],
        scratch_types=[
            pltpu.VMEM((MAXN * LANES,), jnp.float32),
            pltpu.VMEM((MAXN * LANES,), jnp.float32),
            pltpu.VMEM((hb * ns,), jnp.int32),
            [pltpu.VMEM((ns, cb), jnp.float32)] * 2,
            [pltpu.VMEM((ns, cb), jnp.float32)] * 2,
            [pltpu.SemaphoreType.DMA] * 2,
        ],
    )
    def lookup(tab_r_hbm, tab_i_hbm, pos_hbm, out_r_hbm, out_i_hbm,
               tab_r, tab_i, pos_v, re_v, im_v, osem):
        wid = lax.axis_index("s") * info.num_cores + lax.axis_index("c")
        base = wid * rows_w
        pltpu.sync_copy(tab_r_hbm, tab_r)
        pltpu.sync_copy(tab_i_hbm, tab_i)
        lane = lax.iota(jnp.int32, LANES)
        lane_s = lane * ns

        out_dmas = [None, None]
        for g in range(nchunks):
            j = g % 2
            rv, iv = re_v[j], im_v[j]
            if out_dmas[j] is not None:
                for d in out_dmas[j]:
                    d.wait()
            for h in range(cb // hb):
                pltpu.sync_copy(
                    pos_hbm.at[pl.ds((base + g * cb + h * hb) * ns, hb * ns)],
                    pos_v)
                c0 = h * hb

                @plsc.parallel_loop(0, ns * kbh, unroll=16)
                def body(i):
                    s = i // kbh
                    k = i % kbh
                    p = plsc.load_gather(pos_v, [lane_s + (k * LANES * ns + s)])
                    ridx = ((p & (MAXN - 1)) * LANES) + lane
                    rv[s, pl.ds(c0 + k * LANES, LANES)] = (
                        plsc.load_gather(tab_r, [ridx]))
                    iv[s, pl.ds(c0 + k * LANES, LANES)] = (
                        plsc.load_gather(tab_i, [ridx]))

            b0 = base + g * cb
            out_dmas[j] = (
                pltpu.async_copy(rv, out_r_hbm.at[:, pl.ds(b0, cb)], osem[j]),
                pltpu.async_copy(iv, out_i_hbm.at[:, pl.ds(b0, cb)], osem[j]),
            )
        for pair in out_dmas:
            if pair is not None:
                for d in pair:
                    d.wait()

    return lookup


def kernel(pe, pos):
    nb, ns = pos.shape
    tab_r = jnp.repeat(jnp.real(pe).astype(jnp.float32), LANES)
    tab_i = jnp.repeat(jnp.imag(pe).astype(jnp.float32), LANES)
    out_r, out_i = _build_lookup(nb, ns)(tab_r, tab_i, pos.reshape(nb * ns))
    return lax.complex(out_r, out_i).T
